# baseline (speedup 1.0000x reference)
import jax
import jax.numpy as jnp
from jax.experimental import pallas as pl
from jax.experimental.pallas import tpu as pltpu

PADDING_IDX = 0

_STREAMS = 4
_BLOCK_B = 8


def _matmul_kernel(*refs):
    x_refs = refs[:_STREAMS]
    w_ref = refs[_STREAMS]
    o_ref = refs[_STREAMS + 1]
    w = w_ref[...]
    row_ids = jax.lax.broadcasted_iota(jnp.int32, w.shape, 0)
    w = jnp.where(row_ids == PADDING_IDX, 0.0, w).astype(jnp.bfloat16)
    for s in range(_STREAMS):
        for j in range(_BLOCK_B):
            x = x_refs[s][j].astype(jnp.bfloat16)
            o_ref[s * _BLOCK_B + j] = jnp.dot(
                x, w, preferred_element_type=jnp.float32
            )


def _make_in_spec(s, l, v):
    return pl.BlockSpec((_BLOCK_B, l, v), lambda i: (_STREAMS * i + s, 0, 0))


def kernel(input, weight):
    b, l, v = input.shape
    d = weight.shape[1]
    group = _STREAMS * _BLOCK_B
    grid = (b // group,)
    return pl.pallas_call(
        _matmul_kernel,
        grid=grid,
        in_specs=[_make_in_spec(s, l, v) for s in range(_STREAMS)]
        + [pl.BlockSpec((v, d), lambda i: (0, 0))],
        out_specs=pl.BlockSpec((group, l, d), lambda i: (i, 0, 0)),
        out_shape=jax.ShapeDtypeStruct((b, l, d), jnp.float32),
        compiler_params=pltpu.CompilerParams(
            dimension_semantics=("arbitrary",),
        ),
    )(*([input] * _STREAMS), weight)
